# Initial kernel scaffold; baseline (speedup 1.0000x reference)
import jax
import jax.numpy as jnp
from jax.experimental import pallas as pl


def kernel(node_features, edge_index, edge_types, distances, query_idx, rel_emb, Wg, bg, W1, b1, W2, b2):
    raise NotImplementedError("write your pallas kernel here")


# speedup vs baseline: 1.8997x; 1.8997x over previous
import functools

import jax
import jax.numpy as jnp
from jax import lax
from jax.experimental import pallas as pl
from jax.experimental.pallas import tpu as pltpu
from jax.experimental.pallas import tpu_sc as plsc

NCORES = 2
NSUB = 16
NW = NCORES * NSUB
CHUNK = 128


@functools.lru_cache(maxsize=None)
def _make_sc_scatter(dt, npad, nch, gather):
    rows_per_tile = npad // NSUB
    mesh = plsc.VectorSubcoreMesh(core_axis_name="c", subcore_axis_name="s",
                                  num_cores=NCORES)

    @functools.partial(
        pl.kernel,
        mesh=mesh,
        out_type=jax.ShapeDtypeStruct((NCORES, npad, dt), jnp.float32),
        scratch_types=[
            pltpu.VMEM_SHARED((npad, dt), jnp.float32),
            pltpu.VMEM((CHUNK,), jnp.int32),
            pltpu.VMEM((CHUNK,), jnp.int32),
            pltpu.VMEM((CHUNK, dt), jnp.float32),
            pltpu.SemaphoreType.DMA,
        ],
    )
    def sc_scatter(table, gidx, sidx, zeros, out, acc, g_v, s_v, rows_v, sem):
        c = lax.axis_index("c")
        s = lax.axis_index("s")
        wid = s * NCORES + c
        base = s * rows_per_tile
        pltpu.sync_copy(zeros.at[pl.ds(base, rows_per_tile)],
                        acc.at[pl.ds(base, rows_per_tile)])
        if not gather:
            pltpu.sync_copy(table, rows_v)
        plsc.subcore_barrier()

        def chunk(j, carry):
            pltpu.sync_copy(sidx.at[wid, j], s_v)
            if gather:
                pltpu.sync_copy(gidx.at[wid, j], g_v)
                pltpu.async_copy(table.at[g_v], rows_v, sem).wait()
            pltpu.sync_copy(rows_v, acc.at[s_v], add=True)
            return carry

        lax.fori_loop(0, nch, chunk, 0)
        plsc.subcore_barrier()
        pltpu.sync_copy(acc.at[pl.ds(base, rows_per_tile)],
                        out.at[c, pl.ds(base, rows_per_tile)])

    return sc_scatter


def _dense_layer(x, ax, ar, ac, w, b):
    n, d = x.shape
    blk = 1000
    grid = n // blk

    def body(x_ref, ax_ref, ar_ref, ac_ref, w_ref, b_ref, o_ref):
        a = ax_ref[0] + ax_ref[1]
        r = ar_ref[0] + ar_ref[1]
        cnt = ac_ref[0, :, 0:1] + ac_ref[1, :, 0:1]
        agg = (a + r) / jnp.maximum(cnt, 1.0)
        h = x_ref[...] + agg
        y = jnp.dot(h, w_ref[...], preferred_element_type=jnp.float32)
        o_ref[...] = jnp.maximum(y + b_ref[...], 0.0)

    return pl.pallas_call(
        body,
        grid=(grid,),
        in_specs=[
            pl.BlockSpec((blk, d), lambda i: (i, 0)),
            pl.BlockSpec((NCORES, blk, d), lambda i: (0, i, 0)),
            pl.BlockSpec((NCORES, blk, d), lambda i: (0, i, 0)),
            pl.BlockSpec((NCORES, blk, d), lambda i: (0, i, 0)),
            pl.BlockSpec((d, d), lambda i: (0, 0)),
            pl.BlockSpec((1, d), lambda i: (0, 0)),
        ],
        out_specs=pl.BlockSpec((blk, d), lambda i: (i, 0)),
        out_shape=jax.ShapeDtypeStruct((n, d), jnp.float32),
    )(x, ax, ar, ac, w, b)


def _pool_mlp(x, q, w1, b1, w2, b2):
    n, d = x.shape

    def body(x_ref, q_ref, w1_ref, b1_ref, w2_ref, b2_ref, o_ref):
        g = jnp.mean(x_ref[...], axis=0, keepdims=True)
        comb = jnp.concatenate([q_ref[...], g], axis=1)
        h = jnp.dot(comb, w1_ref[...], preferred_element_type=jnp.float32)
        h = jnp.maximum(h + b1_ref[...], 0.0)
        y = jnp.dot(h, w2_ref[...], preferred_element_type=jnp.float32)
        o_ref[...] = y + b2_ref[...]

    out = pl.pallas_call(
        body,
        out_shape=jax.ShapeDtypeStruct((1, d), jnp.float32),
    )(x, q, w1, b1, w2, b2)
    return out.reshape(d)


def kernel(node_features, edge_index, edge_types, distances, query_idx,
           rel_emb, Wg, bg, W1, b1, W2, b2):
    n, d = node_features.shape
    e = edge_index.shape[1]
    r = rel_emb.shape[0]
    nlayers = Wg.shape[0]

    npad = ((n + 1 + NSUB * 8 - 1) // (NSUB * 8)) * (NSUB * 8)
    per = NW * CHUNK
    e2 = 2 * e
    e2p = ((e2 + per - 1) // per) * per
    nch = e2p // per

    ei = edge_index.astype(jnp.int32)
    src, dst = ei[0], ei[1]
    dist = distances.astype(jnp.int32)
    d_src, d_dst = dist[src], dist[dst]
    mf = d_src == d_dst + 1
    mb = d_dst == d_src + 1

    sidx = jnp.concatenate([jnp.where(mf, dst, n), jnp.where(mb, src, n)])
    gidx = jnp.concatenate([src, dst])
    et = edge_types.astype(jnp.int32)
    tidx = jnp.concatenate([et, et])
    pad = e2p - e2
    sidx = jnp.pad(sidx, (0, pad), constant_values=n).reshape(NW, nch, CHUNK)
    gidx = jnp.pad(gidx, (0, pad)).reshape(NW, nch, CHUNK)
    tidx = jnp.pad(tidx, (0, pad)).reshape(NW, nch, CHUNK)

    zeros_x = jnp.zeros((npad, d), jnp.float32)
    ones_blk = jnp.ones((CHUNK, d), jnp.float32)

    re_pass = _make_sc_scatter(d, npad, nch, True)
    cnt_pass = _make_sc_scatter(d, npad, nch, False)
    x_pass = re_pass

    ar = re_pass(rel_emb, tidx, sidx, zeros_x)
    ac = cnt_pass(ones_blk, tidx, sidx, zeros_x)

    x = node_features
    for l in range(nlayers):
        ax = x_pass(x, gidx, sidx, zeros_x)
        x = _dense_layer(x, ax, ar, ac, Wg[l], bg[l].reshape(1, d))

    q = x[query_idx][None]
    return _pool_mlp(x, q, W1, b1.reshape(1, d), W2, b2.reshape(1, d))
